# baseline (speedup 1.0000x reference)
import jax
import jax.numpy as jnp
from jax.experimental import pallas as pl
from jax.experimental.pallas import tpu as pltpu

NUM_EXPERTS = 8
TOP_K = 2
HIDDEN = 1024
INTER = 2816
TOKENS = 2048

PAIRS = TOKENS * TOP_K
BN = 256
NB = INTER // BN
NSTEP = NUM_EXPERTS * NB
RC = 256
GU = 16
CU = 8
ROWS = PAIRS + 128 + RC


def _moe_body(counts_ref, starts_ref, tok_ref, pos_ref, w01_ref,
              hs_ref, gu_hbm, wd_hbm, out_ref,
              x_s, acc_s, wgb, wub, wdb, sg, su, sd):
    e = pl.program_id(0)
    n = pl.program_id(1)
    g = e * NB + n
    cnt = counts_ref[e]
    start = pl.multiple_of(starts_ref[e], GU)
    nch = (cnt + RC - 1) // RC

    def copies(ee, nn, slot):
        return (
            pltpu.make_async_copy(
                gu_hbm.at[ee, :, pl.ds(nn * BN, BN)], wgb.at[slot], sg.at[slot]),
            pltpu.make_async_copy(
                gu_hbm.at[ee, :, pl.ds(INTER + nn * BN, BN)], wub.at[slot], su.at[slot]),
            pltpu.make_async_copy(
                wd_hbm.at[ee, pl.ds(nn * BN, BN), :], wdb.at[slot], sd.at[slot]),
        )

    @pl.when(g == 0)
    def _prime():
        for c in copies(0, 0, 0):
            c.start()

    @pl.when(g + 1 < NSTEP)
    def _issue_next():
        gn = g + 1
        en = gn // NB
        nn = gn - en * NB
        for c in copies(en, nn, gn % 2):
            c.start()

    for c in copies(e, n, g % 2):
        c.wait()

    @pl.when(n == 0)
    def _gather():
        def gbody(t, _):
            base = start + t * GU
            rows = [hs_ref[pl.ds(tok_ref[base + u], 1), :] for u in range(GU)]
            x_s[pl.ds(base, GU), :] = jnp.concatenate(
                rows, axis=0).astype(jnp.bfloat16)
            return 0
        jax.lax.fori_loop(0, (cnt + GU - 1) // GU, gbody, 0)

    slot = g % 2
    wg = wgb[slot]
    wu = wub[slot]
    wd = wdb[slot]

    def chunk(j, add):
        r0 = start + j * RC
        xj = x_s[pl.ds(r0, RC), :].astype(jnp.float32)
        gate = jnp.dot(xj, wg, preferred_element_type=jnp.float32,
                       precision=jax.lax.Precision.DEFAULT)
        up = jnp.dot(xj, wu, preferred_element_type=jnp.float32,
                     precision=jax.lax.Precision.DEFAULT)
        h = gate * jax.nn.sigmoid(gate) * up
        y = jnp.dot(h, wd, preferred_element_type=jnp.float32,
                    precision=jax.lax.Precision.DEFAULT)
        if add:
            acc_s[pl.ds(r0, RC), :] += y
        else:
            acc_s[pl.ds(r0, RC), :] = y
        return 0

    @pl.when(n == 0)
    def _first():
        jax.lax.fori_loop(0, nch, lambda j, c: chunk(j, False), 0)

    @pl.when(n != 0)
    def _rest():
        jax.lax.fori_loop(0, nch, lambda j, c: chunk(j, True), 0)

    @pl.when((e == NUM_EXPERTS - 1) & (n == NB - 1))
    def _combine():
        def cbody(t, _):
            base = t * CU
            for u in range(CU):
                row = base + u
                p0 = pos_ref[2 * row]
                p1 = pos_ref[2 * row + 1]
                w0 = w01_ref[2 * row]
                w1 = w01_ref[2 * row + 1]
                out_ref[pl.ds(row, 1), :] = (
                    w0 * acc_s[pl.ds(p0, 1), :] + w1 * acc_s[pl.ds(p1, 1), :])
            return 0
        jax.lax.fori_loop(0, TOKENS // CU, cbody, 0)


@jax.jit
def kernel(hidden_states, routing_weights, selected_experts, gate_up_proj, down_proj):
    flat_e = selected_experts.reshape(-1)
    order = jnp.argsort(flat_e)
    sorted_e = flat_e[order]
    counts = jnp.bincount(flat_e, length=NUM_EXPERTS).astype(jnp.int32)
    starts = (jnp.cumsum(counts) - counts).astype(jnp.int32)
    p = jnp.zeros((), jnp.int32)
    ps = []
    for ee in range(NUM_EXPERTS):
        ps.append(p)
        p = ((p + counts[ee]) + GU - 1) // GU * GU
    pad_starts = jnp.stack(ps).astype(jnp.int32)
    dest = pad_starts[sorted_e] + (
        jnp.arange(PAIRS, dtype=jnp.int32) - starts[sorted_e])
    tok_pad = jnp.zeros((ROWS,), jnp.int32).at[dest].set(
        (order // TOP_K).astype(jnp.int32))
    pos = jnp.zeros((PAIRS,), jnp.int32).at[order].set(dest)
    w01 = routing_weights.reshape(-1)

    grid_spec = pltpu.PrefetchScalarGridSpec(
        num_scalar_prefetch=5,
        grid=(NUM_EXPERTS, NB),
        in_specs=[
            pl.BlockSpec((TOKENS, HIDDEN), lambda e, n, *_: (0, 0)),
            pl.BlockSpec(memory_space=pl.ANY),
            pl.BlockSpec(memory_space=pl.ANY),
        ],
        out_specs=pl.BlockSpec((TOKENS, HIDDEN), lambda e, n, *_: (0, 0)),
        scratch_shapes=[
            pltpu.VMEM((ROWS, HIDDEN), jnp.bfloat16),
            pltpu.VMEM((ROWS, HIDDEN), jnp.float32),
            pltpu.VMEM((2, HIDDEN, BN), jnp.float32),
            pltpu.VMEM((2, HIDDEN, BN), jnp.float32),
            pltpu.VMEM((2, BN, HIDDEN), jnp.float32),
            pltpu.SemaphoreType.DMA((2,)),
            pltpu.SemaphoreType.DMA((2,)),
            pltpu.SemaphoreType.DMA((2,)),
        ],
    )

    out = pl.pallas_call(
        _moe_body,
        grid_spec=grid_spec,
        out_shape=jax.ShapeDtypeStruct((TOKENS, HIDDEN), jnp.float32),
        compiler_params=pltpu.CompilerParams(
            dimension_semantics=("arbitrary", "arbitrary"),
        ),
    )(counts, pad_starts, tok_pad, pos, w01,
      hidden_states, gate_up_proj, down_proj)
    return out


# speedup vs baseline: 1.0198x; 1.0198x over previous
import jax
import jax.numpy as jnp
from jax.experimental import pallas as pl
from jax.experimental.pallas import tpu as pltpu

NUM_EXPERTS = 8
TOP_K = 2
HIDDEN = 1024
INTER = 2816
TOKENS = 2048

PAIRS = TOKENS * TOP_K
BN = 256
NB = INTER // BN
RC = 256


def _moe_body(counts_ref, starts_ref, tok_ref, w_ref,
              hs_ref, wg_ref, wu_ref, wd_ref, out_ref,
              x_s, acc_s):
    e = pl.program_id(0)
    n = pl.program_id(1)
    cnt = counts_ref[e]
    start = starts_ref[e]
    nch = (cnt + RC - 1) // RC

    @pl.when((e == 0) & (n == 0))
    def _init_out():
        out_ref[...] = jnp.zeros_like(out_ref)

    @pl.when(n == 0)
    def _gather():
        def gbody(i, _):
            tok = tok_ref[start + i]
            x_s[pl.ds(i, 1), :] = hs_ref[pl.ds(tok, 1), :]
            return 0
        jax.lax.fori_loop(0, cnt, gbody, 0)

    wg = wg_ref[0].astype(jnp.bfloat16)
    wu = wu_ref[0].astype(jnp.bfloat16)
    wd = wd_ref[0].astype(jnp.bfloat16)

    def chunk(j, add):
        xj = x_s[pl.ds(j * RC, RC), :].astype(jnp.bfloat16)
        gate = jnp.dot(xj, wg, preferred_element_type=jnp.float32)
        up = jnp.dot(xj, wu, preferred_element_type=jnp.float32)
        h = (gate * jax.nn.sigmoid(gate) * up).astype(jnp.bfloat16)
        y = jnp.dot(h, wd, preferred_element_type=jnp.float32)
        if add:
            acc_s[pl.ds(j * RC, RC), :] += y
        else:
            acc_s[pl.ds(j * RC, RC), :] = y
        return 0

    @pl.when(n == 0)
    def _first():
        jax.lax.fori_loop(0, nch, lambda j, c: chunk(j, False), 0)

    @pl.when(n != 0)
    def _rest():
        jax.lax.fori_loop(0, nch, lambda j, c: chunk(j, True), 0)

    @pl.when(n == NB - 1)
    def _scatter():
        def sbody(i, _):
            tok = tok_ref[start + i]
            wv = w_ref[start + i]
            out_ref[pl.ds(tok, 1), :] += wv * acc_s[pl.ds(i, 1), :]
            return 0
        jax.lax.fori_loop(0, cnt, sbody, 0)


@jax.jit
def kernel(hidden_states, routing_weights, selected_experts, gate_up_proj, down_proj):
    flat_e = selected_experts.reshape(-1)
    order = jnp.argsort(flat_e)
    sorted_tok = (order // TOP_K).astype(jnp.int32)
    sorted_w = routing_weights.reshape(-1)[order]
    counts = jnp.bincount(flat_e, length=NUM_EXPERTS).astype(jnp.int32)
    starts = (jnp.cumsum(counts) - counts).astype(jnp.int32)

    grid_spec = pltpu.PrefetchScalarGridSpec(
        num_scalar_prefetch=4,
        grid=(NUM_EXPERTS, NB),
        in_specs=[
            pl.BlockSpec((TOKENS, HIDDEN), lambda e, n, *_: (0, 0)),
            pl.BlockSpec((1, HIDDEN, BN), lambda e, n, *_: (e, 0, n)),
            pl.BlockSpec((1, HIDDEN, BN), lambda e, n, *_: (e, 0, n + NB)),
            pl.BlockSpec((1, BN, HIDDEN), lambda e, n, *_: (e, n, 0)),
        ],
        out_specs=pl.BlockSpec((TOKENS, HIDDEN), lambda e, n, *_: (0, 0)),
        scratch_shapes=[
            pltpu.VMEM((PAIRS, HIDDEN), jnp.float32),
            pltpu.VMEM((PAIRS, HIDDEN), jnp.float32),
        ],
    )

    out = pl.pallas_call(
        _moe_body,
        grid_spec=grid_spec,
        out_shape=jax.ShapeDtypeStruct((TOKENS, HIDDEN), jnp.float32),
        compiler_params=pltpu.CompilerParams(
            dimension_semantics=("arbitrary", "arbitrary"),
        ),
    )(counts, starts, sorted_tok, sorted_w,
      hidden_states, gate_up_proj, gate_up_proj, down_proj)
    return out


# speedup vs baseline: 1.0211x; 1.0012x over previous
import jax
import jax.numpy as jnp
from jax.experimental import pallas as pl
from jax.experimental.pallas import tpu as pltpu

NUM_EXPERTS = 8
TOP_K = 2
HIDDEN = 1024
INTER = 2816
TOKENS = 2048

PAIRS = TOKENS * TOP_K
BN = 256
NB = INTER // BN
RC = 256


def _moe_body(counts_ref, starts_ref, tok_ref, w_ref,
              hs_ref, wg_ref, wu_ref, wd_ref, out_ref,
              x_s, acc_s, wg16_s, wu16_s, wd16_s):
    e = pl.program_id(0)
    n = pl.program_id(1)
    cnt = counts_ref[e]
    start = starts_ref[e]
    nch = (cnt + RC - 1) // RC

    @pl.when((e == 0) & (n == 0))
    def _init_out():
        out_ref[...] = jnp.zeros_like(out_ref)

    @pl.when(n == 0)
    def _gather():
        def gbody(i, _):
            tok = tok_ref[start + i]
            x_s[pl.ds(i, 1), :] = hs_ref[pl.ds(tok, 1), :]
            return 0
        jax.lax.fori_loop(0, cnt, gbody, 0)

    for k in range(HIDDEN // RC):
        wg16_s[pl.ds(k * RC, RC), :] = (
            wg_ref[0, pl.ds(k * RC, RC), :].astype(jnp.bfloat16))
        wu16_s[pl.ds(k * RC, RC), :] = (
            wu_ref[0, pl.ds(k * RC, RC), :].astype(jnp.bfloat16))
    wd16_s[...] = wd_ref[0].astype(jnp.bfloat16)

    def chunk(j, add):
        xj = x_s[pl.ds(j * RC, RC), :].astype(jnp.bfloat16)
        gate = jnp.dot(xj, wg16_s[...], preferred_element_type=jnp.float32)
        up = jnp.dot(xj, wu16_s[...], preferred_element_type=jnp.float32)
        h = (gate * jax.nn.sigmoid(gate) * up).astype(jnp.bfloat16)
        y = jnp.dot(h, wd16_s[...], preferred_element_type=jnp.float32)
        if add:
            acc_s[pl.ds(j * RC, RC), :] += y
        else:
            acc_s[pl.ds(j * RC, RC), :] = y
        return 0

    @pl.when(n == 0)
    def _first():
        jax.lax.fori_loop(0, nch, lambda j, c: chunk(j, False), 0)

    @pl.when(n != 0)
    def _rest():
        jax.lax.fori_loop(0, nch, lambda j, c: chunk(j, True), 0)

    @pl.when(n == NB - 1)
    def _scatter():
        def sbody(i, _):
            tok = tok_ref[start + i]
            wv = w_ref[start + i]
            out_ref[pl.ds(tok, 1), :] += wv * acc_s[pl.ds(i, 1), :]
            return 0
        jax.lax.fori_loop(0, cnt, sbody, 0)


@jax.jit
def kernel(hidden_states, routing_weights, selected_experts, gate_up_proj, down_proj):
    flat_e = selected_experts.reshape(-1)
    order = jnp.argsort(flat_e)
    sorted_tok = (order // TOP_K).astype(jnp.int32)
    sorted_w = routing_weights.reshape(-1)[order]
    counts = jnp.bincount(flat_e, length=NUM_EXPERTS).astype(jnp.int32)
    starts = (jnp.cumsum(counts) - counts).astype(jnp.int32)

    grid_spec = pltpu.PrefetchScalarGridSpec(
        num_scalar_prefetch=4,
        grid=(NUM_EXPERTS, NB),
        in_specs=[
            pl.BlockSpec((TOKENS, HIDDEN), lambda e, n, *_: (0, 0)),
            pl.BlockSpec((1, HIDDEN, BN), lambda e, n, *_: (e, 0, n)),
            pl.BlockSpec((1, HIDDEN, BN), lambda e, n, *_: (e, 0, n + NB)),
            pl.BlockSpec((1, BN, HIDDEN), lambda e, n, *_: (e, n, 0)),
        ],
        out_specs=pl.BlockSpec((TOKENS, HIDDEN), lambda e, n, *_: (0, 0)),
        scratch_shapes=[
            pltpu.VMEM((PAIRS, HIDDEN), jnp.float32),
            pltpu.VMEM((PAIRS, HIDDEN), jnp.float32),
            pltpu.VMEM((HIDDEN, BN), jnp.bfloat16),
            pltpu.VMEM((HIDDEN, BN), jnp.bfloat16),
            pltpu.VMEM((BN, HIDDEN), jnp.bfloat16),
        ],
    )

    out = pl.pallas_call(
        _moe_body,
        grid_spec=grid_spec,
        out_shape=jax.ShapeDtypeStruct((TOKENS, HIDDEN), jnp.float32),
        compiler_params=pltpu.CompilerParams(
            dimension_semantics=("arbitrary", "arbitrary"),
        ),
    )(counts, starts, sorted_tok, sorted_w,
      hidden_states, gate_up_proj, gate_up_proj, down_proj)
    return out
